# Initial kernel scaffold; baseline (speedup 1.0000x reference)
import jax
import jax.numpy as jnp
from jax.experimental import pallas as pl


def kernel(H, indice, W):
    raise NotImplementedError("write your pallas kernel here")


# speedup vs baseline: 177.8210x; 177.8210x over previous
import functools

import jax
import jax.numpy as jnp
from jax import lax
from jax.experimental import pallas as pl
from jax.experimental.pallas import tpu as pltpu
from jax.experimental.pallas import tpu_sc as plsc


def _sc_gather(H2d, idx_flat, N, F):
    TOT = idx_flat.shape[0]
    D = H2d.shape[1]
    info = plsc.get_sparse_core_info()
    NC, NS, L = info.num_cores, info.num_subcores, info.num_lanes
    NW = NC * NS
    per_w = TOT // NW
    CHUNK = 128
    n_chunks = per_w // CHUNK
    assert per_w % CHUNK == 0 and per_w % L == 0 and TOT % NW == 0

    mesh = plsc.VectorSubcoreMesh(core_axis_name="c", subcore_axis_name="s")

    @functools.partial(
        pl.kernel,
        out_type=jax.ShapeDtypeStruct((TOT, D), jnp.float32),
        mesh=mesh,
        compiler_params=pltpu.CompilerParams(use_tc_tiling_on_sc=False),
        scratch_types=[
            pltpu.VMEM((per_w,), jnp.int32),
            pltpu.VMEM((per_w,), jnp.int32),
            pltpu.VMEM((CHUNK, D), jnp.float32),
            pltpu.SemaphoreType.DMA,
        ],
    )
    def k(h_hbm, idx_hbm, out_hbm, idxraw_v, rowidx_v, rows_v, gsem):
        wid = lax.axis_index("s") * NC + lax.axis_index("c")
        base = wid * per_w
        pltpu.sync_copy(idx_hbm.at[pl.ds(base, per_w)], idxraw_v)

        def compute_rows(t, carry):
            p = base + t * L + lax.broadcasted_iota(jnp.int32, (L,), 0)
            b_of_p = lax.div(p, F)
            rowidx_v[pl.ds(t * L, L)] = idxraw_v[pl.ds(t * L, L)] + b_of_p * N
            return carry

        lax.fori_loop(0, per_w // L, compute_rows, 0)

        def gather_chunk(c, carry):
            idx_slice = rowidx_v.at[pl.ds(c * CHUNK, CHUNK)]
            pltpu.async_copy(h_hbm.at[idx_slice], rows_v, gsem).wait()
            pltpu.sync_copy(rows_v, out_hbm.at[pl.ds(base + c * CHUNK, CHUNK)])
            return carry

        lax.fori_loop(0, n_chunks, gather_chunk, 0)

    return k(H2d, idx_flat)


def _tc_mean(H):
    B, N, D = H.shape
    Bb = 128

    def body(h_ref, o_ref):
        o_ref[...] = jnp.sum(h_ref[...], axis=1) * (1.0 / N)

    return pl.pallas_call(
        body,
        grid=(B // Bb,),
        in_specs=[pl.BlockSpec((Bb, N, D), lambda i: (i, 0, 0))],
        out_specs=pl.BlockSpec((Bb, D), lambda i: (i, 0)),
        out_shape=jax.ShapeDtypeStruct((B, D), jnp.float32),
    )(H)


def _tc_dense(g2d, meanv, W):
    B, FD = g2d.shape
    D = meanv.shape[1]

    Bb = 512
    dims = (((1,), (0,)), ((), ()))

    def body(g_ref, m_ref, w_ref, o_ref):
        g = jnp.maximum(g_ref[...], 0.0)
        acc = lax.dot_general(g, w_ref[0:FD, :], dims,
                              preferred_element_type=jnp.float32)
        acc = acc + lax.dot_general(m_ref[...], w_ref[FD:FD + D, :], dims,
                                    preferred_element_type=jnp.float32)
        o_ref[...] = jnp.maximum(acc, 0.0)

    return pl.pallas_call(
        body,
        grid=(B // Bb,),
        in_specs=[
            pl.BlockSpec((Bb, FD), lambda i: (i, 0)),
            pl.BlockSpec((Bb, D), lambda i: (i, 0)),
            pl.BlockSpec((FD + D, D), lambda i: (0, 0)),
        ],
        out_specs=pl.BlockSpec((Bb, D), lambda i: (i, 0)),
        out_shape=jax.ShapeDtypeStruct((B, D), jnp.float32),
    )(g2d, meanv, W)


def kernel(H, indice, W):
    B, N, D = H.shape
    F = indice.shape[1]
    idx_flat = indice.astype(jnp.int32).reshape(B * F)
    H2d = H.reshape(B * N, D)
    gathered = _sc_gather(H2d, idx_flat, N, F)
    meanv = _tc_mean(H)
    out = _tc_dense(gathered.reshape(B, F * D), meanv, W)
    return out[:, None, :]


# speedup vs baseline: 180.7484x; 1.0165x over previous
import functools

import jax
import jax.numpy as jnp
from jax import lax
from jax.experimental import pallas as pl
from jax.experimental.pallas import tpu as pltpu
from jax.experimental.pallas import tpu_sc as plsc


def _sc_gather(H2d, idx_flat, N, F):
    TOT = idx_flat.shape[0]
    D = H2d.shape[1]
    info = plsc.get_sparse_core_info()
    NC, NS, L = info.num_cores, info.num_subcores, info.num_lanes
    NW = NC * NS
    per_w = TOT // NW
    CHUNK = 128
    n_chunks = per_w // CHUNK
    assert per_w % CHUNK == 0 and per_w % L == 0 and TOT % NW == 0

    mesh = plsc.VectorSubcoreMesh(core_axis_name="c", subcore_axis_name="s")

    @functools.partial(
        pl.kernel,
        out_type=jax.ShapeDtypeStruct((TOT, D), jnp.float32),
        mesh=mesh,
        compiler_params=pltpu.CompilerParams(use_tc_tiling_on_sc=False),
        scratch_types=[
            pltpu.VMEM((per_w,), jnp.int32),
            pltpu.VMEM((per_w,), jnp.int32),
            pltpu.VMEM((CHUNK, D), jnp.float32),
            pltpu.SemaphoreType.DMA,
        ],
    )
    def k(h_hbm, idx_hbm, out_hbm, idxraw_v, rowidx_v, rows_v, gsem):
        wid = lax.axis_index("s") * NC + lax.axis_index("c")
        base = wid * per_w
        pltpu.sync_copy(idx_hbm.at[pl.ds(base, per_w)], idxraw_v)

        def compute_rows(t, carry):
            p = base + t * L + lax.broadcasted_iota(jnp.int32, (L,), 0)
            b_of_p = lax.div(p, F)
            rowidx_v[pl.ds(t * L, L)] = idxraw_v[pl.ds(t * L, L)] + b_of_p * N
            return carry

        lax.fori_loop(0, per_w // L, compute_rows, 0)

        def gather_chunk(c, carry):
            idx_slice = rowidx_v.at[pl.ds(c * CHUNK, CHUNK)]
            pltpu.async_copy(h_hbm.at[idx_slice], rows_v, gsem).wait()
            pltpu.sync_copy(rows_v, out_hbm.at[pl.ds(base + c * CHUNK, CHUNK)])
            return carry

        lax.fori_loop(0, n_chunks, gather_chunk, 0)

    return k(H2d, idx_flat)


def _tc_mean(Hw, N, D):
    B, NW, W = Hw.shape
    fold = W // D
    Bb = 128

    def body(h_ref, o_ref):
        s = jnp.sum(h_ref[...], axis=1)
        acc = s[:, 0:D]
        for k in range(1, fold):
            acc = acc + s[:, k * D:(k + 1) * D]
        o_ref[...] = acc * (1.0 / N)

    return pl.pallas_call(
        body,
        grid=(B // Bb,),
        in_specs=[pl.BlockSpec((Bb, NW, W), lambda i: (i, 0, 0))],
        out_specs=pl.BlockSpec((Bb, D), lambda i: (i, 0)),
        out_shape=jax.ShapeDtypeStruct((B, D), jnp.float32),
    )(Hw)


def _tc_dense(g2d, meanv, W):
    B, FD = g2d.shape
    D = meanv.shape[1]

    Bb = 512
    dims = (((1,), (0,)), ((), ()))

    def body(g_ref, m_ref, w_ref, o_ref):
        g = jnp.maximum(g_ref[...], 0.0)
        acc = lax.dot_general(g, w_ref[0:FD, :], dims,
                              preferred_element_type=jnp.float32)
        acc = acc + lax.dot_general(m_ref[...], w_ref[FD:FD + D, :], dims,
                                    preferred_element_type=jnp.float32)
        o_ref[...] = jnp.maximum(acc, 0.0)

    return pl.pallas_call(
        body,
        grid=(B // Bb,),
        in_specs=[
            pl.BlockSpec((Bb, FD), lambda i: (i, 0)),
            pl.BlockSpec((Bb, D), lambda i: (i, 0)),
            pl.BlockSpec((FD + D, D), lambda i: (0, 0)),
        ],
        out_specs=pl.BlockSpec((Bb, D), lambda i: (i, 0)),
        out_shape=jax.ShapeDtypeStruct((B, D), jnp.float32),
    )(g2d, meanv, W)


def kernel(H, indice, W):
    B, N, D = H.shape
    F = indice.shape[1]
    idx_flat = indice.astype(jnp.int32).reshape(B * F)
    H2d = H.reshape(B * N, D)
    gathered = _sc_gather(H2d, idx_flat, N, F)
    meanv = _tc_mean(H.reshape(B, N * D // 128, 128), N, D)
    out = _tc_dense(gathered.reshape(B, F * D), meanv, W)
    return out[:, None, :]
